# baseline (device time: 280849 ns/iter reference)
import jax
import jax.numpy as jnp
from jax import lax
from jax.experimental import pallas as pl
from jax.experimental.pallas import tpu as pltpu

N_DEV = 8
N_TOK = 4096
D_IN = 1024
D_OUT = 2048
N_EXP = 32
E_LOCAL = 4
C = 192
R = E_LOCAL * C
CHUNK = N_TOK // N_DEV
HALF = CHUNK // 2


def _ar_body(pexp_ref, y_ref, out_ref, stage_r, stage_l,
             send_sem_r, send_sem_l, recv_sem_r, recv_sem_l,
             credit_r, credit_l):
    my = lax.axis_index("i")
    left = (my - 1) % N_DEV
    right = (my + 1) % N_DEV

    barrier_sem = pltpu.get_barrier_semaphore()
    for nbr in (left, right):
        pl.semaphore_signal(
            barrier_sem, inc=1, device_id=(nbr,),
            device_id_type=pl.DeviceIdType.MESH,
        )
    pl.semaphore_wait(barrier_sem, 2)

    pl.semaphore_signal(credit_r, inc=1, device_id=(left,),
                        device_id_type=pl.DeviceIdType.MESH)
    pl.semaphore_signal(credit_l, inc=1, device_id=(right,),
                        device_id_type=pl.DeviceIdType.MESH)

    out_ref[...] = jnp.dot(
        pexp_ref[...], y_ref[...], preferred_element_type=jnp.float32
    ).astype(jnp.bfloat16)

    def hop(s, phase):
        if phase == "rs":
            send_cr = (my - s) % N_DEV
            recv_cr = (my - s - 1) % N_DEV
            send_cl = (my + s) % N_DEV
            recv_cl = (my + s + 1) % N_DEV
        else:
            send_cr = (my + 1 - s) % N_DEV
            send_cl = (my - 1 + s) % N_DEV

        pl.semaphore_wait(credit_r, 1)
        pl.semaphore_wait(credit_l, 1)
        if phase == "rs":
            dst_r, dst_l = stage_r, stage_l
        else:
            dst_r = out_ref.at[pl.ds(send_cr * CHUNK, HALF), :]
            dst_l = out_ref.at[pl.ds(send_cl * CHUNK + HALF, HALF), :]
        rd_r = pltpu.make_async_remote_copy(
            src_ref=out_ref.at[pl.ds(send_cr * CHUNK, HALF), :],
            dst_ref=dst_r,
            send_sem=send_sem_r,
            recv_sem=recv_sem_r,
            device_id=(right,),
            device_id_type=pl.DeviceIdType.MESH,
        )
        rd_l = pltpu.make_async_remote_copy(
            src_ref=out_ref.at[pl.ds(send_cl * CHUNK + HALF, HALF), :],
            dst_ref=dst_l,
            send_sem=send_sem_l,
            recv_sem=recv_sem_l,
            device_id=(left,),
            device_id_type=pl.DeviceIdType.MESH,
        )
        rd_r.start()
        rd_l.start()
        rd_r.wait()
        rd_l.wait()
        if phase == "rs":
            out_ref[pl.ds(recv_cr * CHUNK, HALF), :] += stage_r[...]
            out_ref[pl.ds(recv_cl * CHUNK + HALF, HALF), :] += stage_l[...]
        pl.semaphore_signal(credit_r, inc=1, device_id=(left,),
                            device_id_type=pl.DeviceIdType.MESH)
        pl.semaphore_signal(credit_l, inc=1, device_id=(right,),
                            device_id_type=pl.DeviceIdType.MESH)

    for s in range(N_DEV - 1):
        hop(s, "rs")
    for s in range(N_DEV - 1):
        hop(s, "ag")

    pl.semaphore_wait(credit_r, 1)
    pl.semaphore_wait(credit_l, 1)


def _pallas_expand_allreduce(pexp, y):
    return pl.pallas_call(
        _ar_body,
        out_shape=jax.ShapeDtypeStruct((N_TOK, D_OUT), jnp.bfloat16),
        in_specs=[
            pl.BlockSpec(memory_space=pltpu.VMEM),
            pl.BlockSpec(memory_space=pltpu.VMEM),
        ],
        out_specs=pl.BlockSpec(memory_space=pltpu.VMEM),
        scratch_shapes=[
            pltpu.VMEM((HALF, D_OUT), jnp.bfloat16),
            pltpu.VMEM((HALF, D_OUT), jnp.bfloat16),
            pltpu.SemaphoreType.DMA,
            pltpu.SemaphoreType.DMA,
            pltpu.SemaphoreType.DMA,
            pltpu.SemaphoreType.DMA,
            pltpu.SemaphoreType.REGULAR,
            pltpu.SemaphoreType.REGULAR,
        ],
        compiler_params=pltpu.CompilerParams(collective_id=0),
    )(pexp, y)


def kernel(x, router_W, route_idx, expert_W):
    del router_W
    my = lax.axis_index("i")
    e0 = my * E_LOCAL

    ids = route_idx[:, 0]
    xb = x.astype(jnp.bfloat16)

    onehot = ids[:, None] == jnp.arange(N_EXP, dtype=ids.dtype)[None, :]
    pos_mat = jnp.cumsum(onehot.astype(jnp.int32), axis=0) - 1
    tok_pos = jnp.sum(jnp.where(onehot, pos_mat, 0), axis=1)

    local_slot = (ids - e0) * C + tok_pos
    valid = (ids >= e0) & (ids < e0 + E_LOCAL) & (tok_pos < C)
    slot = jnp.where(valid, local_slot, -1)
    Pexp = (slot[:, None] == jnp.arange(R, dtype=slot.dtype)[None, :]).astype(jnp.bfloat16)

    buf = jax.lax.dot_general(
        Pexp, xb, dimension_numbers=(((0,), (0,)), ((), ())),
        preferred_element_type=jnp.bfloat16,
    )
    y = jax.lax.dot_general(
        buf.reshape(E_LOCAL, C, D_IN),
        expert_W.astype(jnp.bfloat16),
        dimension_numbers=(((2,), (1,)), ((0,), (0,))),
        preferred_element_type=jnp.float32,
    ).astype(jnp.bfloat16).reshape(R, D_OUT)

    return _pallas_expand_allreduce(Pexp, y)


# device time: 275483 ns/iter; 1.0195x vs baseline; 1.0195x over previous
import jax
import jax.numpy as jnp
from jax import lax
from jax.experimental import pallas as pl
from jax.experimental.pallas import tpu as pltpu

N_DEV = 8
N_TOK = 4096
D_IN = 1024
D_OUT = 2048
N_EXP = 32
E_LOCAL = 4
C = 192
R = E_LOCAL * C
CHUNK = N_TOK // N_DEV
HALF = CHUNK // 2


def _ar_body(p_ref, out_ref, stage_r, stage_l,
             send_sem_r, send_sem_l, recv_sem_r, recv_sem_l,
             credit_r, credit_l):
    my = lax.axis_index("i")
    left = (my - 1) % N_DEV
    right = (my + 1) % N_DEV

    barrier_sem = pltpu.get_barrier_semaphore()
    for nbr in (left, right):
        pl.semaphore_signal(
            barrier_sem, inc=1, device_id=(nbr,),
            device_id_type=pl.DeviceIdType.MESH,
        )
    pl.semaphore_wait(barrier_sem, 2)

    pl.semaphore_signal(credit_r, inc=1, device_id=(left,),
                        device_id_type=pl.DeviceIdType.MESH)
    pl.semaphore_signal(credit_l, inc=1, device_id=(right,),
                        device_id_type=pl.DeviceIdType.MESH)

    out_ref[...] = p_ref[...]

    def hop(s, phase):
        if phase == "rs":
            send_cr = (my - s) % N_DEV
            recv_cr = (my - s - 1) % N_DEV
            send_cl = (my + s) % N_DEV
            recv_cl = (my + s + 1) % N_DEV
        else:
            send_cr = (my + 1 - s) % N_DEV
            send_cl = (my - 1 + s) % N_DEV

        pl.semaphore_wait(credit_r, 1)
        pl.semaphore_wait(credit_l, 1)
        if phase == "rs":
            dst_r, dst_l = stage_r, stage_l
        else:
            dst_r = out_ref.at[pl.ds(send_cr * CHUNK, HALF), :]
            dst_l = out_ref.at[pl.ds(send_cl * CHUNK + HALF, HALF), :]
        rd_r = pltpu.make_async_remote_copy(
            src_ref=out_ref.at[pl.ds(send_cr * CHUNK, HALF), :],
            dst_ref=dst_r,
            send_sem=send_sem_r,
            recv_sem=recv_sem_r,
            device_id=(right,),
            device_id_type=pl.DeviceIdType.MESH,
        )
        rd_l = pltpu.make_async_remote_copy(
            src_ref=out_ref.at[pl.ds(send_cl * CHUNK + HALF, HALF), :],
            dst_ref=dst_l,
            send_sem=send_sem_l,
            recv_sem=recv_sem_l,
            device_id=(left,),
            device_id_type=pl.DeviceIdType.MESH,
        )
        rd_r.start()
        rd_l.start()
        rd_r.wait()
        rd_l.wait()
        if phase == "rs":
            out_ref[pl.ds(recv_cr * CHUNK, HALF), :] += stage_r[...]
            out_ref[pl.ds(recv_cl * CHUNK + HALF, HALF), :] += stage_l[...]
        pl.semaphore_signal(credit_r, inc=1, device_id=(left,),
                            device_id_type=pl.DeviceIdType.MESH)
        pl.semaphore_signal(credit_l, inc=1, device_id=(right,),
                            device_id_type=pl.DeviceIdType.MESH)

    for s in range(N_DEV - 1):
        hop(s, "rs")
    for s in range(N_DEV - 1):
        hop(s, "ag")

    pl.semaphore_wait(credit_r, 1)
    pl.semaphore_wait(credit_l, 1)


def _pallas_allreduce(partial):
    return pl.pallas_call(
        _ar_body,
        out_shape=jax.ShapeDtypeStruct((N_TOK, D_OUT), jnp.bfloat16),
        in_specs=[pl.BlockSpec(memory_space=pltpu.VMEM)],
        out_specs=pl.BlockSpec(memory_space=pltpu.VMEM),
        scratch_shapes=[
            pltpu.VMEM((HALF, D_OUT), jnp.bfloat16),
            pltpu.VMEM((HALF, D_OUT), jnp.bfloat16),
            pltpu.SemaphoreType.DMA,
            pltpu.SemaphoreType.DMA,
            pltpu.SemaphoreType.DMA,
            pltpu.SemaphoreType.DMA,
            pltpu.SemaphoreType.REGULAR,
            pltpu.SemaphoreType.REGULAR,
        ],
        compiler_params=pltpu.CompilerParams(collective_id=0),
    )(partial)


def kernel(x, router_W, route_idx, expert_W):
    del router_W
    my = lax.axis_index("i")
    e0 = my * E_LOCAL

    ids = route_idx[:, 0]
    xb = x.astype(jnp.bfloat16)

    onehot = ids[:, None] == jnp.arange(N_EXP, dtype=ids.dtype)[None, :]
    pos_mat = jnp.cumsum(onehot.astype(jnp.int32), axis=0) - 1
    tok_pos = jnp.sum(jnp.where(onehot, pos_mat, 0), axis=1)

    local_slot = (ids - e0) * C + tok_pos
    valid = (ids >= e0) & (ids < e0 + E_LOCAL) & (tok_pos < C)
    slot = jnp.where(valid, local_slot, -1)
    Pexp = (slot[:, None] == jnp.arange(R, dtype=slot.dtype)[None, :]).astype(jnp.bfloat16)

    buf = jax.lax.dot_general(
        Pexp, xb, dimension_numbers=(((0,), (0,)), ((), ())),
        preferred_element_type=jnp.bfloat16,
    )
    y = jax.lax.dot_general(
        buf.reshape(E_LOCAL, C, D_IN),
        expert_W.astype(jnp.bfloat16),
        dimension_numbers=(((2,), (1,)), ((0,), (0,))),
        preferred_element_type=jnp.float32,
    ).astype(jnp.bfloat16).reshape(R, D_OUT)

    partial = jax.lax.dot_general(
        Pexp, y, dimension_numbers=(((1,), (0,)), ((), ())),
        preferred_element_type=jnp.bfloat16,
    )

    return _pallas_allreduce(partial)


# device time: 263860 ns/iter; 1.0644x vs baseline; 1.0440x over previous
import jax
import jax.numpy as jnp
from jax import lax
from jax.experimental import pallas as pl
from jax.experimental.pallas import tpu as pltpu

N_DEV = 8
N_TOK = 4096
D_IN = 1024
D_OUT = 2048
N_EXP = 32
E_LOCAL = 4
C = 192
R = E_LOCAL * C
CHUNK = N_TOK // N_DEV
HALF = CHUNK // 2
QUART = CHUNK // 4
N_STREAMS = 4


def _ar_body(p_ref, out_ref, stages, send_sems, recv_sems, credits):
    my = lax.axis_index("i")
    left = (my - 1) % N_DEV
    right = (my + 1) % N_DEV

    barrier_sem = pltpu.get_barrier_semaphore()
    for nbr in (left, right):
        pl.semaphore_signal(
            barrier_sem, inc=1, device_id=(nbr,),
            device_id_type=pl.DeviceIdType.MESH,
        )
    pl.semaphore_wait(barrier_sem, 2)

    streams = []
    for k in range(N_STREAMS):
        is_right = k < 2
        streams.append(dict(
            q=k,
            to_peer=right if is_right else left,
            from_peer=left if is_right else right,
            sgn=1 if is_right else -1,
        ))

    for st in streams:
        pl.semaphore_signal(credits.at[st["q"]], inc=1,
                            device_id=(st["from_peer"],),
                            device_id_type=pl.DeviceIdType.MESH)

    out_ref[...] = p_ref[...]

    def sub(c, q):
        return out_ref.at[pl.ds(c * CHUNK + q * QUART, QUART), :]

    def hop(s, phase):
        rdmas = []
        for st in streams:
            q, sgn = st["q"], st["sgn"]
            if phase == "rs":
                send_c = (my - sgn * s) % N_DEV
                dst = stages.at[q]
            else:
                send_c = (my + sgn * (1 - s)) % N_DEV
                dst = sub(send_c, q)
            rd = pltpu.make_async_remote_copy(
                src_ref=sub(send_c, q),
                dst_ref=dst,
                send_sem=send_sems.at[q],
                recv_sem=recv_sems.at[q],
                device_id=(st["to_peer"],),
                device_id_type=pl.DeviceIdType.MESH,
            )
            pl.semaphore_wait(credits.at[q], 1)
            rd.start()
            rdmas.append(rd)
        for st, rd in zip(streams, rdmas):
            q, sgn = st["q"], st["sgn"]
            rd.wait()
            if phase == "rs":
                recv_c = (my - sgn * (s + 1)) % N_DEV
                out_ref[pl.ds(recv_c * CHUNK + q * QUART, QUART), :] += (
                    stages[q, :, :]
                )
            pl.semaphore_signal(credits.at[q], inc=1,
                                device_id=(st["from_peer"],),
                                device_id_type=pl.DeviceIdType.MESH)

    for s in range(N_DEV - 1):
        hop(s, "rs")
    for s in range(N_DEV - 1):
        hop(s, "ag")

    for st in streams:
        pl.semaphore_wait(credits.at[st["q"]], 1)


def _pallas_allreduce(partial):
    return pl.pallas_call(
        _ar_body,
        out_shape=jax.ShapeDtypeStruct((N_TOK, D_OUT), jnp.bfloat16),
        in_specs=[pl.BlockSpec(memory_space=pltpu.VMEM)],
        out_specs=pl.BlockSpec(memory_space=pltpu.VMEM),
        scratch_shapes=[
            pltpu.VMEM((N_STREAMS, QUART, D_OUT), jnp.bfloat16),
            pltpu.SemaphoreType.DMA((N_STREAMS,)),
            pltpu.SemaphoreType.DMA((N_STREAMS,)),
            pltpu.SemaphoreType.REGULAR((N_STREAMS,)),
        ],
        compiler_params=pltpu.CompilerParams(collective_id=0),
    )(partial)


def kernel(x, router_W, route_idx, expert_W):
    del router_W
    my = lax.axis_index("i")
    e0 = my * E_LOCAL

    ids = route_idx[:, 0]
    xb = x.astype(jnp.bfloat16)

    onehot = ids[:, None] == jnp.arange(N_EXP, dtype=ids.dtype)[None, :]
    pos_mat = jnp.cumsum(onehot.astype(jnp.int32), axis=0) - 1
    tok_pos = jnp.sum(jnp.where(onehot, pos_mat, 0), axis=1)

    local_slot = (ids - e0) * C + tok_pos
    valid = (ids >= e0) & (ids < e0 + E_LOCAL) & (tok_pos < C)
    slot = jnp.where(valid, local_slot, -1)
    Pexp = (slot[:, None] == jnp.arange(R, dtype=slot.dtype)[None, :]).astype(jnp.bfloat16)

    buf = jax.lax.dot_general(
        Pexp, xb, dimension_numbers=(((0,), (0,)), ((), ())),
        preferred_element_type=jnp.bfloat16,
    )
    y = jax.lax.dot_general(
        buf.reshape(E_LOCAL, C, D_IN),
        expert_W.astype(jnp.bfloat16),
        dimension_numbers=(((2,), (1,)), ((0,), (0,))),
        preferred_element_type=jnp.float32,
    ).astype(jnp.bfloat16).reshape(R, D_OUT)

    partial = jax.lax.dot_general(
        Pexp, y, dimension_numbers=(((1,), (0,)), ((), ())),
        preferred_element_type=jnp.bfloat16,
    )

    return _pallas_allreduce(partial)


# device time: 255155 ns/iter; 1.1007x vs baseline; 1.0341x over previous
import jax
import jax.numpy as jnp
from jax import lax
from jax.experimental import pallas as pl
from jax.experimental.pallas import tpu as pltpu

N_DEV = 8
N_TOK = 4096
D_IN = 1024
D_OUT = 2048
N_EXP = 32
E_LOCAL = 4
C = 192
R = E_LOCAL * C
CHUNK = N_TOK // N_DEV
HALF = CHUNK // 2
QUART = CHUNK // 4
N_STREAMS = 4


def _ar_body(p_ref, out_ref, stages, send_sems, recv_sems, credits):
    my = lax.axis_index("i")
    left = (my - 1) % N_DEV
    right = (my + 1) % N_DEV

    barrier_sem = pltpu.get_barrier_semaphore()
    for nbr in (left, right):
        pl.semaphore_signal(
            barrier_sem, inc=1, device_id=(nbr,),
            device_id_type=pl.DeviceIdType.MESH,
        )
    pl.semaphore_wait(barrier_sem, 2)

    streams = []
    for k in range(N_STREAMS):
        is_right = k < 2
        streams.append(dict(
            q=k,
            to_peer=right if is_right else left,
            from_peer=left if is_right else right,
            sgn=1 if is_right else -1,
        ))

    for st in streams:
        pl.semaphore_signal(credits.at[st["q"]], inc=1,
                            device_id=(st["from_peer"],),
                            device_id_type=pl.DeviceIdType.MESH)

    del p_ref

    def sub(c, q):
        return out_ref.at[pl.ds(c * CHUNK + q * QUART, QUART), :]

    def hop(s, phase):
        rdmas = []
        for st in streams:
            q, sgn = st["q"], st["sgn"]
            if phase == "rs":
                send_c = (my - sgn * s) % N_DEV
                dst = stages.at[q]
            else:
                send_c = (my + sgn * (1 - s)) % N_DEV
                dst = sub(send_c, q)
            rd = pltpu.make_async_remote_copy(
                src_ref=sub(send_c, q),
                dst_ref=dst,
                send_sem=send_sems.at[q],
                recv_sem=recv_sems.at[q],
                device_id=(st["to_peer"],),
                device_id_type=pl.DeviceIdType.MESH,
            )
            pl.semaphore_wait(credits.at[q], 1)
            rd.start()
            rdmas.append(rd)
        for st, rd in zip(streams, rdmas):
            q, sgn = st["q"], st["sgn"]
            rd.wait()
            if phase == "rs":
                recv_c = (my - sgn * (s + 1)) % N_DEV
                out_ref[pl.ds(recv_c * CHUNK + q * QUART, QUART), :] += (
                    stages[q, :, :]
                )
            pl.semaphore_signal(credits.at[q], inc=1,
                                device_id=(st["from_peer"],),
                                device_id_type=pl.DeviceIdType.MESH)

    for s in range(N_DEV - 1):
        hop(s, "rs")
    for s in range(N_DEV - 1):
        hop(s, "ag")

    for st in streams:
        pl.semaphore_wait(credits.at[st["q"]], 1)


def _pallas_allreduce(partial):
    return pl.pallas_call(
        _ar_body,
        out_shape=jax.ShapeDtypeStruct((N_TOK, D_OUT), jnp.bfloat16),
        in_specs=[pl.BlockSpec(memory_space=pltpu.VMEM)],
        out_specs=pl.BlockSpec(memory_space=pltpu.VMEM),
        scratch_shapes=[
            pltpu.VMEM((N_STREAMS, QUART, D_OUT), jnp.bfloat16),
            pltpu.SemaphoreType.DMA((N_STREAMS,)),
            pltpu.SemaphoreType.DMA((N_STREAMS,)),
            pltpu.SemaphoreType.REGULAR((N_STREAMS,)),
        ],
        input_output_aliases={0: 0},
        compiler_params=pltpu.CompilerParams(collective_id=0),
    )(partial)


def kernel(x, router_W, route_idx, expert_W):
    del router_W
    my = lax.axis_index("i")
    e0 = my * E_LOCAL

    ids = route_idx[:, 0]
    xb = x.astype(jnp.bfloat16)

    onehot = ids[:, None] == jnp.arange(N_EXP, dtype=ids.dtype)[None, :]
    pos_mat = jnp.cumsum(onehot.astype(jnp.int32), axis=0) - 1
    tok_pos = jnp.sum(jnp.where(onehot, pos_mat, 0), axis=1)

    local_slot = (ids - e0) * C + tok_pos
    valid = (ids >= e0) & (ids < e0 + E_LOCAL) & (tok_pos < C)
    slot = jnp.where(valid, local_slot, -1)
    Pexp = (slot[:, None] == jnp.arange(R, dtype=slot.dtype)[None, :]).astype(jnp.bfloat16)

    buf = jax.lax.dot_general(
        Pexp, xb, dimension_numbers=(((0,), (0,)), ((), ())),
        preferred_element_type=jnp.bfloat16,
    )
    y = jax.lax.dot_general(
        buf.reshape(E_LOCAL, C, D_IN),
        expert_W.astype(jnp.bfloat16),
        dimension_numbers=(((2,), (1,)), ((0,), (0,))),
        preferred_element_type=jnp.float32,
    ).astype(jnp.bfloat16).reshape(R, D_OUT)

    partial = jax.lax.dot_general(
        Pexp, y, dimension_numbers=(((1,), (0,)), ((), ())),
        preferred_element_type=jnp.bfloat16,
    )

    return _pallas_allreduce(partial)
